# Initial kernel scaffold; baseline (speedup 1.0000x reference)
import jax
import jax.numpy as jnp
from jax.experimental import pallas as pl


def kernel(x, edge_index, W1, att_l1, att_r1, b1, W2, att_l2, att_r2, b2):
    raise NotImplementedError("write your pallas kernel here")


# speedup vs baseline: 15.2164x; 15.2164x over previous
import functools

import jax
import jax.numpy as jnp
from jax import lax
from jax.experimental import pallas as pl
from jax.experimental.pallas import tpu as pltpu
from jax.experimental.pallas import tpu_sc as plsc

N_NODES = 10000
NPAD = 10240
D = 128
NC = 2
NS = 16
NHALF = NPAD // NC
UROWS = 5248
GARBAGE = NHALF
CHUNK = 128
ROWS_PER_TILE = UROWS // NS
WB_CHUNKS = (128, 128, 72)
L = 16


def _lin_body(x_ref, w_ref, attl_ref, attr_ref, xl_ref, al_ref, ar_ref):
    xl = jnp.dot(x_ref[...], w_ref[...], preferred_element_type=jnp.float32)
    xl_ref[...] = xl
    al_ref[...] = jnp.dot(xl, attl_ref[...], preferred_element_type=jnp.float32)
    ar_ref[...] = jnp.dot(xl, attr_ref[...], preferred_element_type=jnp.float32)


def _lin_call(xp, W, attl, attr):
    return pl.pallas_call(
        _lin_body,
        out_shape=[
            jax.ShapeDtypeStruct((NPAD, D), jnp.float32),
            jax.ShapeDtypeStruct((NPAD, 1), jnp.float32),
            jax.ShapeDtypeStruct((NPAD, 1), jnp.float32),
        ],
    )(xp, W, attl, attr)


def _merge_u(u_ref, dp_ref, b_ref):
    denom = jnp.sum(dp_ref[...], axis=0)
    u = jnp.concatenate([u_ref[0, :NHALF], u_ref[1, :NHALF]], axis=0)
    return u / (denom[:, None] + 1e-16) + b_ref[...]


def _mid_body(u_ref, dp_ref, b_ref, w_ref, attl_ref, attr_ref,
              xl_ref, al_ref, ar_ref):
    h = _merge_u(u_ref, dp_ref, b_ref)
    h = jnp.where(h > 0, h, jnp.exp(h) - 1.0)
    row = lax.broadcasted_iota(jnp.int32, (NPAD, 1), 0)
    h = jnp.where(row < N_NODES, h, 0.0)
    xl = jnp.dot(h, w_ref[...], preferred_element_type=jnp.float32)
    xl_ref[...] = xl
    al_ref[...] = jnp.dot(xl, attl_ref[...], preferred_element_type=jnp.float32)
    ar_ref[...] = jnp.dot(xl, attr_ref[...], preferred_element_type=jnp.float32)


def _mid_call(u, dp, b, W, attl, attr):
    return pl.pallas_call(
        _mid_body,
        out_shape=[
            jax.ShapeDtypeStruct((NPAD, D), jnp.float32),
            jax.ShapeDtypeStruct((NPAD, 1), jnp.float32),
            jax.ShapeDtypeStruct((NPAD, 1), jnp.float32),
        ],
    )(u, dp, b, W, attl, attr)


def _out_body(u_ref, dp_ref, b_ref, o_ref):
    z = _merge_u(u_ref, dp_ref, b_ref)
    m = jnp.max(z, axis=1, keepdims=True)
    ez = jnp.exp(z - m)
    o_ref[...] = z - m - jnp.log(jnp.sum(ez, axis=1, keepdims=True))


def _out_call(u, dp, b):
    return pl.pallas_call(
        _out_body,
        out_shape=jax.ShapeDtypeStruct((NPAD, D), jnp.float32),
    )(u, dp, b)


def _make_edge_kernel(cpt):
    mesh = plsc.VectorSubcoreMesh(core_axis_name="c", subcore_axis_name="s")

    @functools.partial(
        pl.kernel,
        out_type=[
            jax.ShapeDtypeStruct((NC, UROWS, D), jnp.float32),
            jax.ShapeDtypeStruct((NS, NPAD), jnp.float32),
        ],
        mesh=mesh,
        compiler_params=pltpu.CompilerParams(needs_layout_passes=False),
        scratch_types=[
            pltpu.VMEM((NPAD,), jnp.float32),
            pltpu.VMEM((NPAD,), jnp.float32),
            pltpu.VMEM((NPAD,), jnp.float32),
            pltpu.VMEM((cpt, CHUNK), jnp.int32),
            pltpu.VMEM((cpt, CHUNK), jnp.int32),
            pltpu.VMEM((1, CHUNK), jnp.int32),
            pltpu.VMEM((CHUNK,), jnp.float32),
            pltpu.VMEM((CHUNK, D), jnp.float32),
            pltpu.VMEM_SHARED((UROWS, D), jnp.float32),
            pltpu.SemaphoreType.DMA,
        ],
    )
    def edge_kernel(xl_hbm, al_hbm, ar_hbm, src_hbm, dst_hbm,
                    u_hbm, dp_hbm,
                    al_v, ar_v, den_v, src_v, dst_v, dl_v, e_v, rows_v,
                    u_sh, sem):
        cid = lax.axis_index("c")
        sid = lax.axis_index("s")
        rbase = cid * NHALF

        pltpu.sync_copy(al_hbm, al_v)
        pltpu.sync_copy(ar_hbm, ar_v)
        pltpu.sync_copy(src_hbm.at[sid], src_v)
        pltpu.sync_copy(dst_hbm.at[sid], dst_v)

        zeros16 = jnp.zeros((L,), jnp.float32)

        def zden(i, carry):
            den_v[pl.ds(i * L, L)] = zeros16
            return carry
        lax.fori_loop(0, NPAD // L, zden, 0)

        def zrow(r, carry):
            for t in range(D // L):
                rows_v[r, pl.ds(t * L, L)] = zeros16
            return carry
        lax.fori_loop(0, CHUNK, zrow, 0)

        base = sid * ROWS_PER_TILE
        off = 0
        for n in WB_CHUNKS:
            pltpu.sync_copy(rows_v.at[pl.ds(0, n)],
                            u_sh.at[pl.ds(base + off, n)])
            off += n
        plsc.subcore_barrier()

        def chunk_body(j, carry):
            pltpu.async_copy(xl_hbm.at[src_v.at[j]], rows_v, sem).wait()
            for t in range(CHUNK // L):
                sl = pl.ds(t * L, L)
                si = src_v[j, sl]
                di = dst_v[j, sl]
                a = plsc.load_gather(al_v, [si]) + plsc.load_gather(ar_v, [di])
                a = jnp.where(a >= 0.0, a, 0.2 * a)
                ee = jnp.exp(a)
                plsc.addupdate_scatter(den_v, [di], ee)
                e_v[sl] = ee
                loc = di - rbase
                ok = (loc >= 0) & (loc < NHALF)
                dl_v[0, sl] = jnp.where(ok, loc, GARBAGE)

            def scale(r, c2):
                coef = plsc.load_gather(e_v, [jnp.zeros((L,), jnp.int32) + r])
                for t in range(D // L):
                    sl = pl.ds(t * L, L)
                    rows_v[r, sl] = rows_v[r, sl] * coef
                return c2
            lax.fori_loop(0, CHUNK, scale, 0)

            pltpu.sync_copy(rows_v, u_sh.at[dl_v.at[0]], add=True)
            return carry
        lax.fori_loop(0, cpt, chunk_body, 0)

        plsc.subcore_barrier()
        off = 0
        for n in WB_CHUNKS:
            pltpu.sync_copy(u_sh.at[pl.ds(base + off, n)],
                            rows_v.at[pl.ds(0, n)])
            pltpu.sync_copy(rows_v.at[pl.ds(0, n)],
                            u_hbm.at[cid, pl.ds(base + off, n)])
            off += n
        @pl.when(cid == 0)
        def _():
            pltpu.sync_copy(den_v, dp_hbm.at[sid])

    return edge_kernel


def kernel(x, edge_index, W1, att_l1, att_r1, b1, W2, att_l2, att_r2, b2):
    n_edges = edge_index.shape[1]
    cpt = -(-n_edges // (NS * CHUNK))
    epad = NS * cpt * CHUNK

    src = edge_index[0].astype(jnp.int32)
    dst = edge_index[1].astype(jnp.int32)
    pad = jnp.full((epad - n_edges,), N_NODES, jnp.int32)
    srcp = jnp.concatenate([src, pad]).reshape(NS, cpt, CHUNK)
    dstp = jnp.concatenate([dst, pad]).reshape(NS, cpt, CHUNK)
    xp = jnp.pad(x, ((0, NPAD - x.shape[0]), (0, 0)))

    edge_call = _make_edge_kernel(cpt)

    xl1, al1, ar1 = _lin_call(xp, W1, att_l1.reshape(D, 1), att_r1.reshape(D, 1))
    u1, dp1 = edge_call(xl1, al1.reshape(NPAD), ar1.reshape(NPAD), srcp, dstp)
    xl2, al2, ar2 = _mid_call(u1, dp1, b1.reshape(1, D), W2,
                              att_l2.reshape(D, 1), att_r2.reshape(D, 1))
    u2, dp2 = edge_call(xl2, al2.reshape(NPAD), ar2.reshape(NPAD), srcp, dstp)
    out = _out_call(u2, dp2, b2.reshape(1, D))
    return out[:N_NODES]
